# baseline (speedup 1.0000x reference)
import functools

import jax
import jax.numpy as jnp
from jax import lax
from jax.experimental import pallas as pl
from jax.experimental.pallas import tpu as pltpu
from jax.experimental.pallas import tpu_sc as plsc

N = 10000
E = 320000
NET = 4
D_IN = 128
H = 64

NPAD = 10240
EPAD = 327680
EPT = EPAD // 16
RPT = NPAD // 16
NROW = EPAD // 128
RWT = NROW // 16
DUMP = N
KB1 = 512
KB2 = 1024
BR = 2048
NB = NPAD // BR

_MESH = plsc.VectorSubcoreMesh(core_axis_name="c", subcore_axis_name="s",
                               num_cores=2, num_subcores=16)
_SC_PARAMS = pltpu.CompilerParams(use_tc_tiling_on_sc=False)


@functools.partial(
    pl.kernel,
    out_type=jax.ShapeDtypeStruct((NET * NPAD,), jnp.float32),
    mesh=_MESH,
    compiler_params=_SC_PARAMS,
    scratch_types=[
        pltpu.VMEM((16,), jnp.float32),
        pltpu.VMEM((128,), jnp.float32),
        pltpu.VMEM((RPT,), jnp.float32),
        pltpu.VMEM((8, 128), jnp.int32),
        pltpu.VMEM_SHARED((NPAD,), jnp.float32),
    ],
)
def _sc_deg(dst3, out, stage, ones_v, zrow_v, didx, acc):
    c = lax.axis_index("c")
    s = lax.axis_index("s")
    stage[...] = jnp.full((16,), 1.0, jnp.float32)
    for j in range(8):
        ones_v[pl.ds(j * 16, 16)] = stage[...]
    stage[...] = jnp.zeros((16,), jnp.float32)

    def zz(j, _):
        zrow_v[pl.ds(j * 16, 16)] = stage[...]
        return 0
    lax.fori_loop(0, RPT // 16, zz, 0)

    for nl in range(2):
        net = 2 * c + nl
        pltpu.sync_copy(zrow_v, acc.at[pl.ds(s * RPT, RPT)])
        plsc.subcore_barrier()
        rbase = net * NROW + s * RWT

        def batch(b, _):
            pltpu.sync_copy(dst3.at[pl.ds(rbase + b * 8, 8)], didx)
            for j in range(8):
                pltpu.sync_copy(ones_v, acc.at[didx.at[j]], add=True)
            return 0
        lax.fori_loop(0, RWT // 8, batch, 0)
        plsc.subcore_barrier()
        pltpu.sync_copy(acc.at[pl.ds(s * RPT, RPT)],
                        out.at[pl.ds(net * NPAD + s * RPT, RPT)])
        plsc.subcore_barrier()


def _make_sc_agg(width, kb):
    nrow_b = kb // 128
    nbatch = EPT // kb
    npair = nbatch // 2

    @functools.partial(
        pl.kernel,
        out_type=jax.ShapeDtypeStruct((NET * NPAD, width), jnp.float32),
        mesh=_MESH,
        compiler_params=_SC_PARAMS,
        scratch_types=[
            pltpu.VMEM((kb,), jnp.int32),
            pltpu.VMEM((kb,), jnp.int32),
            pltpu.VMEM((nrow_b, 128), jnp.int32),
            pltpu.VMEM((nrow_b, 128), jnp.int32),
            pltpu.VMEM((kb, width), jnp.float32),
            pltpu.VMEM((kb, width), jnp.float32),
            pltpu.VMEM_SHARED((NPAD, width), jnp.float32),
            pltpu.SemaphoreType.DMA,
            pltpu.SemaphoreType.DMA,
            pltpu.SemaphoreType.DMA,
            pltpu.SemaphoreType.DMA,
        ],
    )
    def _sc_agg(z, srcg, dst3, zrows, out, sidx0, sidx1, didx0, didx1,
                rows0, rows1, acc, gs0, gs1, ss0, ss1):
        c = lax.axis_index("c")
        s = lax.axis_index("s")
        slots = ((sidx0, didx0, rows0, gs0, ss0),
                 (sidx1, didx1, rows1, gs1, ss1))

        for nl in range(2):
            net = 2 * c + nl
            pltpu.sync_copy(zrows, acc.at[pl.ds(s * RPT, RPT)])
            plsc.subcore_barrier()
            ebase = net * EPAD + s * EPT
            rbase = net * NROW + s * RWT

            pltpu.sync_copy(srcg.at[pl.ds(ebase, kb)], sidx0)
            pltpu.async_copy(z.at[sidx0], rows0, gs0)

            def pair(p, _):
                for k in range(2):
                    b = 2 * p + k
                    sidx, didx, rows, gsem, ssem = slots[k]
                    osidx, _, orows, ogsem, ossem = slots[1 - k]
                    pltpu.make_async_copy(z.at[sidx], rows, gsem).wait()
                    pltpu.sync_copy(
                        dst3.at[pl.ds(rbase + b * nrow_b, nrow_b)], didx)
                    for j in range(nrow_b):
                        pltpu.async_copy(rows.at[pl.ds(j * 128, 128)],
                                         acc.at[didx.at[j]], ssem, add=True)
                    @pl.when(b > 0)
                    def _():
                        pltpu.make_async_copy(z.at[pl.ds(0, kb)], orows,
                                              ossem).wait()

                    @pl.when(b + 1 < nbatch)
                    def _():
                        pltpu.sync_copy(
                            srcg.at[pl.ds(ebase + (b + 1) * kb, kb)], osidx)
                        pltpu.async_copy(z.at[osidx], orows, ogsem)
                return 0
            lax.fori_loop(0, npair, pair, 0)
            pltpu.make_async_copy(z.at[pl.ds(0, kb)], rows1, ss1).wait()
            plsc.subcore_barrier()
            pltpu.sync_copy(acc.at[pl.ds(s * RPT, RPT)],
                            out.at[pl.ds(net * NPAD + s * RPT, RPT)])
            plsc.subcore_barrier()

    return _sc_agg


_sc_agg64 = _make_sc_agg(H, KB1)
_sc_agg8 = _make_sc_agg(8, KB2)


def _tc_b_body(f_ref, w1_ref, deg_ref, z_ref, dinv_ref, s1_ref):
    y = jnp.dot(f_ref[...], w1_ref[0], preferred_element_type=jnp.float32)
    dv = lax.rsqrt(deg_ref[0, 0] + 2.0)
    z_ref[0] = dv[:, None] * y
    dinv_ref[0, 0] = dv
    s1_ref[0] = (2.0 * dv * dv)[:, None] * y


def _tc_b(flatten_p, W1, deg):
    return pl.pallas_call(
        _tc_b_body,
        grid=(NET, NB),
        in_specs=[
            pl.BlockSpec((BR, D_IN), lambda i, j: (j, 0)),
            pl.BlockSpec((1, D_IN, H), lambda i, j: (i, 0, 0)),
            pl.BlockSpec((1, 1, BR), lambda i, j: (i, 0, j)),
        ],
        out_specs=[
            pl.BlockSpec((1, BR, H), lambda i, j: (i, j, 0)),
            pl.BlockSpec((1, 1, BR), lambda i, j: (i, 0, j)),
            pl.BlockSpec((1, BR, H), lambda i, j: (i, j, 0)),
        ],
        out_shape=[
            jax.ShapeDtypeStruct((NET, NPAD, H), jnp.float32),
            jax.ShapeDtypeStruct((NET, 1, NPAD), jnp.float32),
            jax.ShapeDtypeStruct((NET, NPAD, H), jnp.float32),
        ],
    )(flatten_p, W1, deg)


def _x_block(agg_ref, dinv_ref, s1_ref, b1_ref):
    dv = dinv_ref[0, 0]
    x = dv[:, None] * agg_ref[0] + s1_ref[0] + b1_ref[0, 0][None, :]
    return jnp.maximum(x, 0.0), dv


def _tc_c1_body(agg_ref, dinv_ref, s1_ref, b1_ref, sum1_ref, sum2_ref):
    j = pl.program_id(1)
    x, _ = _x_block(agg_ref, dinv_ref, s1_ref, b1_ref)
    rows = lax.broadcasted_iota(jnp.int32, (BR, 1), 0) + j * BR
    mask = (rows < N).astype(jnp.float32)
    xm = x * mask

    @pl.when(j == 0)
    def _():
        sum1_ref[...] = jnp.zeros_like(sum1_ref)
        sum2_ref[...] = jnp.zeros_like(sum2_ref)

    sum1_ref[0, 0] += jnp.sum(xm, axis=0)
    sum2_ref[0, 0] += jnp.sum(xm * x, axis=0)


def _tc_c1(agg, dinv, self1, b1r):
    return pl.pallas_call(
        _tc_c1_body,
        grid=(NET, NB),
        in_specs=[
            pl.BlockSpec((1, BR, H), lambda i, j: (i, j, 0)),
            pl.BlockSpec((1, 1, BR), lambda i, j: (i, 0, j)),
            pl.BlockSpec((1, BR, H), lambda i, j: (i, j, 0)),
            pl.BlockSpec((1, 1, H), lambda i, j: (i, 0, 0)),
        ],
        out_specs=[
            pl.BlockSpec((1, 1, H), lambda i, j: (i, 0, 0)),
            pl.BlockSpec((1, 1, H), lambda i, j: (i, 0, 0)),
        ],
        out_shape=[
            jax.ShapeDtypeStruct((NET, 1, H), jnp.float32),
            jax.ShapeDtypeStruct((NET, 1, H), jnp.float32),
        ],
    )(agg, dinv, self1, b1r)


def _tc_c2_body(agg_ref, dinv_ref, s1_ref, b1_ref, g_ref, be_ref, w2_ref,
                sum1_ref, sum2_ref, zd_ref, sd_ref):
    x, dv = _x_block(agg_ref, dinv_ref, s1_ref, b1_ref)
    mean = sum1_ref[0, 0] / float(N)
    var = sum2_ref[0, 0] / float(N) - mean * mean
    scale = lax.rsqrt(var + 1e-5) * g_ref[0, 0]
    xbn = (x - mean[None, :]) * scale[None, :] + be_ref[0, 0][None, :]
    xd = jnp.dot(xbn, w2_ref[0], preferred_element_type=jnp.float32)
    zd_ref[0] = dv[:, None] * xd
    sd_ref[0] = (2.0 * dv * dv)[:, None] * xd


def _tc_c2(agg, dinv, self1, b1r, gr, ber, w2d, sum1, sum2):
    return pl.pallas_call(
        _tc_c2_body,
        grid=(NET, NB),
        in_specs=[
            pl.BlockSpec((1, BR, H), lambda i, j: (i, j, 0)),
            pl.BlockSpec((1, 1, BR), lambda i, j: (i, 0, j)),
            pl.BlockSpec((1, BR, H), lambda i, j: (i, j, 0)),
            pl.BlockSpec((1, 1, H), lambda i, j: (i, 0, 0)),
            pl.BlockSpec((1, 1, H), lambda i, j: (i, 0, 0)),
            pl.BlockSpec((1, 1, H), lambda i, j: (i, 0, 0)),
            pl.BlockSpec((1, H, 8), lambda i, j: (i, 0, 0)),
            pl.BlockSpec((1, 1, H), lambda i, j: (i, 0, 0)),
            pl.BlockSpec((1, 1, H), lambda i, j: (i, 0, 0)),
        ],
        out_specs=[
            pl.BlockSpec((1, BR, 8), lambda i, j: (i, j, 0)),
            pl.BlockSpec((1, BR, 8), lambda i, j: (i, j, 0)),
        ],
        out_shape=[
            jax.ShapeDtypeStruct((NET, NPAD, 8), jnp.float32),
            jax.ShapeDtypeStruct((NET, NPAD, 8), jnp.float32),
        ],
    )(agg, dinv, self1, b1r, gr, ber, w2d, sum1, sum2)


def _tc_d_body(aggd_ref, sd_ref, dinv_ref, b2d_ref, moet_ref, gwt_ref,
               gb_ref, out_ref):
    wl = jnp.dot(gwt_ref[...], moet_ref[...],
                 preferred_element_type=jnp.float32) + gb_ref[...]
    nid = lax.broadcasted_iota(jnp.int32, (8, 1), 0)
    wl = jnp.where(nid < NET, wl, -1e30)
    wm = jnp.max(wl, axis=0, keepdims=True)
    we = jnp.exp(wl - wm)
    w = (we / jnp.sum(we, axis=0, keepdims=True))[0:NET]

    d = dinv_ref[...] * aggd_ref[...] + sd_ref[...] + b2d_ref[...]
    t = jnp.log(1.0 + jnp.exp(-jnp.abs(d)))
    lp1 = jnp.minimum(d, 0.0) - t
    lp0 = -jnp.maximum(d, 0.0) - t
    r0 = jnp.sum(w * lp0, axis=0, keepdims=True)
    r1 = jnp.sum(w * lp1, axis=0, keepdims=True)
    q0 = jnp.sum(w * jnp.exp(lp0), axis=0, keepdims=True)
    q1 = jnp.sum(w * jnp.exp(lp1), axis=0, keepdims=True)
    out_ref[...] = jnp.concatenate(
        [r0, r1, q0, q1, jnp.zeros((4, w.shape[1]), jnp.float32)], axis=0)


def _tc_d(aggd, selfd, dinv2, b2d, moeT, gWT8, gb8):
    return pl.pallas_call(
        _tc_d_body,
        out_shape=jax.ShapeDtypeStruct((8, NPAD), jnp.float32),
    )(aggd, selfd, dinv2, b2d, moeT, gWT8, gb8)


def kernel(features, moe_features, networks, flatten, W1, b1, gamma, beta,
           W2, b2, gW, gb):
    f32 = jnp.float32
    src = networks[:, 0, :].astype(jnp.int32)
    dst = networks[:, 1, :].astype(jnp.int32)
    pad = EPAD - E
    srcp = jnp.pad(src, ((0, 0), (0, pad)), constant_values=DUMP)
    dstp = jnp.pad(dst, ((0, 0), (0, pad)), constant_values=DUMP)
    srcg = (srcp
            + (jnp.arange(NET, dtype=jnp.int32) * NPAD)[:, None]).reshape(-1)
    dst3 = dstp.reshape(NET * NROW, 128)
    flatten_p = jnp.pad(flatten, ((0, NPAD - N), (0, 0)))
    moeT = jnp.pad(moe_features, ((0, NPAD - N), (0, 0))).T
    w2d = (W2[:, :, 1] - W2[:, :, 0]).reshape(NET, H, 1)
    w2d = jnp.pad(w2d, ((0, 0), (0, 0), (0, 7)))
    b2d = (b2[:, 1] - b2[:, 0]).reshape(NET, 1)
    gWT8 = jnp.pad(gW.T, ((0, 4), (0, 0)))
    gb8 = jnp.pad(gb, (0, 4)).reshape(8, 1)
    zeros64 = jnp.zeros((RPT, H), f32)
    zeros8 = jnp.zeros((RPT, 8), f32)

    deg = _sc_deg(dst3).reshape(NET, 1, NPAD)
    z3, dinv, self1 = _tc_b(flatten_p, W1, deg)
    agg1 = _sc_agg64(z3.reshape(NET * NPAD, H), srcg, dst3, zeros64)
    agg1 = agg1.reshape(NET, NPAD, H)
    b1r = b1.reshape(NET, 1, H)
    sum1, sum2 = _tc_c1(agg1, dinv, self1, b1r)
    zd, sd = _tc_c2(agg1, dinv, self1, b1r, gamma.reshape(NET, 1, H),
                    beta.reshape(NET, 1, H), w2d, sum1, sum2)
    aggd = _sc_agg8(zd.reshape(NET * NPAD, 8), srcg, dst3, zeros8)
    aggd = aggd.reshape(NET, NPAD, 8)[:, :, 0]
    outd = _tc_d(aggd, sd[:, :, 0], dinv.reshape(NET, NPAD), b2d, moeT,
                 gWT8, gb8)
    return (jnp.stack([outd[0, :N], outd[1, :N]], axis=1),
            jnp.stack([outd[2, :N], outd[3, :N]], axis=1))


# speedup vs baseline: 1.0985x; 1.0985x over previous
import functools

import jax
import jax.numpy as jnp
from jax import lax
from jax.experimental import pallas as pl
from jax.experimental.pallas import tpu as pltpu
from jax.experimental.pallas import tpu_sc as plsc

N = 10000
E = 320000
NET = 4
D_IN = 128
H = 64

NPAD = 10240
EPAD = 327680
EPT = EPAD // 16
RPT = NPAD // 16
NROW = EPAD // 128
RWT = NROW // 16
DUMP = N
KB1 = 512
KB2 = 2048
BR = 2048
NB = NPAD // BR

_MESH = plsc.VectorSubcoreMesh(core_axis_name="c", subcore_axis_name="s",
                               num_cores=2, num_subcores=16)
_SC_PARAMS = pltpu.CompilerParams(use_tc_tiling_on_sc=False)


@functools.partial(
    pl.kernel,
    out_type=jax.ShapeDtypeStruct((NET * NPAD,), jnp.float32),
    mesh=_MESH,
    compiler_params=_SC_PARAMS,
    scratch_types=[
        pltpu.VMEM((16,), jnp.float32),
        pltpu.VMEM((128,), jnp.float32),
        pltpu.VMEM((RPT,), jnp.float32),
        pltpu.VMEM((8, 128), jnp.int32),
        pltpu.VMEM_SHARED((NPAD,), jnp.float32),
    ],
)
def _sc_deg(dst3, out, stage, ones_v, zrow_v, didx, acc):
    c = lax.axis_index("c")
    s = lax.axis_index("s")
    stage[...] = jnp.full((16,), 1.0, jnp.float32)
    for j in range(8):
        ones_v[pl.ds(j * 16, 16)] = stage[...]
    stage[...] = jnp.zeros((16,), jnp.float32)

    def zz(j, _):
        zrow_v[pl.ds(j * 16, 16)] = stage[...]
        return 0
    lax.fori_loop(0, RPT // 16, zz, 0)

    for nl in range(2):
        net = 2 * c + nl
        pltpu.sync_copy(zrow_v, acc.at[pl.ds(s * RPT, RPT)])
        plsc.subcore_barrier()
        rbase = net * NROW + s * RWT

        def batch(b, _):
            pltpu.sync_copy(dst3.at[pl.ds(rbase + b * 8, 8)], didx)
            for j in range(8):
                pltpu.sync_copy(ones_v, acc.at[didx.at[j]], add=True)
            return 0
        lax.fori_loop(0, RWT // 8, batch, 0)
        plsc.subcore_barrier()
        pltpu.sync_copy(acc.at[pl.ds(s * RPT, RPT)],
                        out.at[pl.ds(net * NPAD + s * RPT, RPT)])
        plsc.subcore_barrier()


def _make_sc_agg(width, kb):
    nrow_b = kb // 128
    nbatch = EPT // kb
    npair = nbatch // 2

    @functools.partial(
        pl.kernel,
        out_type=jax.ShapeDtypeStruct((NET * NPAD, width), jnp.float32),
        mesh=_MESH,
        compiler_params=_SC_PARAMS,
        scratch_types=[
            pltpu.VMEM((kb,), jnp.int32),
            pltpu.VMEM((kb,), jnp.int32),
            pltpu.VMEM((nrow_b, 128), jnp.int32),
            pltpu.VMEM((nrow_b, 128), jnp.int32),
            pltpu.VMEM((kb, width), jnp.float32),
            pltpu.VMEM((kb, width), jnp.float32),
            pltpu.VMEM_SHARED((NPAD, width), jnp.float32),
            pltpu.SemaphoreType.DMA,
            pltpu.SemaphoreType.DMA,
            pltpu.SemaphoreType.DMA,
            pltpu.SemaphoreType.DMA,
        ],
    )
    def _sc_agg(z, srcg, dst3, zrows, out, sidx0, sidx1, didx0, didx1,
                rows0, rows1, acc, gs0, gs1, ss0, ss1):
        c = lax.axis_index("c")
        s = lax.axis_index("s")
        slots = ((sidx0, didx0, rows0, gs0, ss0),
                 (sidx1, didx1, rows1, gs1, ss1))

        for nl in range(2):
            net = 2 * c + nl
            pltpu.sync_copy(zrows, acc.at[pl.ds(s * RPT, RPT)])
            plsc.subcore_barrier()
            ebase = net * EPAD + s * EPT
            rbase = net * NROW + s * RWT

            pltpu.sync_copy(srcg.at[pl.ds(ebase, kb)], sidx0)
            pltpu.async_copy(z.at[sidx0], rows0, gs0)

            def pair(p, _):
                for k in range(2):
                    b = 2 * p + k
                    sidx, didx, rows, gsem, _ = slots[k]
                    osidx, _, orows, ogsem, _ = slots[1 - k]
                    @pl.when(b + 1 < nbatch)
                    def _():
                        pltpu.sync_copy(
                            srcg.at[pl.ds(ebase + (b + 1) * kb, kb)], osidx)
                        pltpu.async_copy(z.at[osidx], orows, ogsem)
                    pltpu.make_async_copy(z.at[sidx], rows, gsem).wait()
                    pltpu.sync_copy(
                        dst3.at[pl.ds(rbase + b * nrow_b, nrow_b)], didx)
                    for j in range(nrow_b):
                        pltpu.sync_copy(rows.at[pl.ds(j * 128, 128)],
                                        acc.at[didx.at[j]], add=True)
                return 0
            lax.fori_loop(0, npair, pair, 0)
            plsc.subcore_barrier()
            pltpu.sync_copy(acc.at[pl.ds(s * RPT, RPT)],
                            out.at[pl.ds(net * NPAD + s * RPT, RPT)])
            plsc.subcore_barrier()

    return _sc_agg


_sc_agg64 = _make_sc_agg(H, KB1)
_sc_agg8 = _make_sc_agg(8, KB2)


def _tc_b_body(f_ref, w1_ref, deg_ref, z_ref, dinv_ref, s1_ref):
    y = jnp.dot(f_ref[...], w1_ref[0], preferred_element_type=jnp.float32)
    dv = lax.rsqrt(deg_ref[0, 0] + 2.0)
    z_ref[0] = dv[:, None] * y
    dinv_ref[0, 0] = dv
    s1_ref[0] = (2.0 * dv * dv)[:, None] * y


def _tc_b(flatten_p, W1, deg):
    return pl.pallas_call(
        _tc_b_body,
        grid=(NET, NB),
        in_specs=[
            pl.BlockSpec((BR, D_IN), lambda i, j: (j, 0)),
            pl.BlockSpec((1, D_IN, H), lambda i, j: (i, 0, 0)),
            pl.BlockSpec((1, 1, BR), lambda i, j: (i, 0, j)),
        ],
        out_specs=[
            pl.BlockSpec((1, BR, H), lambda i, j: (i, j, 0)),
            pl.BlockSpec((1, 1, BR), lambda i, j: (i, 0, j)),
            pl.BlockSpec((1, BR, H), lambda i, j: (i, j, 0)),
        ],
        out_shape=[
            jax.ShapeDtypeStruct((NET, NPAD, H), jnp.float32),
            jax.ShapeDtypeStruct((NET, 1, NPAD), jnp.float32),
            jax.ShapeDtypeStruct((NET, NPAD, H), jnp.float32),
        ],
    )(flatten_p, W1, deg)


def _x_block(agg_ref, dinv_ref, s1_ref, b1_ref):
    dv = dinv_ref[0, 0]
    x = dv[:, None] * agg_ref[0] + s1_ref[0] + b1_ref[0, 0][None, :]
    return jnp.maximum(x, 0.0), dv


def _tc_c1_body(agg_ref, dinv_ref, s1_ref, b1_ref, sum1_ref, sum2_ref):
    j = pl.program_id(1)
    x, _ = _x_block(agg_ref, dinv_ref, s1_ref, b1_ref)
    rows = lax.broadcasted_iota(jnp.int32, (BR, 1), 0) + j * BR
    mask = (rows < N).astype(jnp.float32)
    xm = x * mask

    @pl.when(j == 0)
    def _():
        sum1_ref[...] = jnp.zeros_like(sum1_ref)
        sum2_ref[...] = jnp.zeros_like(sum2_ref)

    sum1_ref[0, 0] += jnp.sum(xm, axis=0)
    sum2_ref[0, 0] += jnp.sum(xm * x, axis=0)


def _tc_c1(agg, dinv, self1, b1r):
    return pl.pallas_call(
        _tc_c1_body,
        grid=(NET, NB),
        in_specs=[
            pl.BlockSpec((1, BR, H), lambda i, j: (i, j, 0)),
            pl.BlockSpec((1, 1, BR), lambda i, j: (i, 0, j)),
            pl.BlockSpec((1, BR, H), lambda i, j: (i, j, 0)),
            pl.BlockSpec((1, 1, H), lambda i, j: (i, 0, 0)),
        ],
        out_specs=[
            pl.BlockSpec((1, 1, H), lambda i, j: (i, 0, 0)),
            pl.BlockSpec((1, 1, H), lambda i, j: (i, 0, 0)),
        ],
        out_shape=[
            jax.ShapeDtypeStruct((NET, 1, H), jnp.float32),
            jax.ShapeDtypeStruct((NET, 1, H), jnp.float32),
        ],
    )(agg, dinv, self1, b1r)


def _tc_c2_body(agg_ref, dinv_ref, s1_ref, b1_ref, g_ref, be_ref, w2_ref,
                sum1_ref, sum2_ref, zd_ref, sd_ref):
    x, dv = _x_block(agg_ref, dinv_ref, s1_ref, b1_ref)
    mean = sum1_ref[0, 0] / float(N)
    var = sum2_ref[0, 0] / float(N) - mean * mean
    scale = lax.rsqrt(var + 1e-5) * g_ref[0, 0]
    xbn = (x - mean[None, :]) * scale[None, :] + be_ref[0, 0][None, :]
    xd = jnp.dot(xbn, w2_ref[0], preferred_element_type=jnp.float32)
    zd_ref[0] = dv[:, None] * xd
    sd_ref[0] = (2.0 * dv * dv)[:, None] * xd


def _tc_c2(agg, dinv, self1, b1r, gr, ber, w2d, sum1, sum2):
    return pl.pallas_call(
        _tc_c2_body,
        grid=(NET, NB),
        in_specs=[
            pl.BlockSpec((1, BR, H), lambda i, j: (i, j, 0)),
            pl.BlockSpec((1, 1, BR), lambda i, j: (i, 0, j)),
            pl.BlockSpec((1, BR, H), lambda i, j: (i, j, 0)),
            pl.BlockSpec((1, 1, H), lambda i, j: (i, 0, 0)),
            pl.BlockSpec((1, 1, H), lambda i, j: (i, 0, 0)),
            pl.BlockSpec((1, 1, H), lambda i, j: (i, 0, 0)),
            pl.BlockSpec((1, H, 8), lambda i, j: (i, 0, 0)),
            pl.BlockSpec((1, 1, H), lambda i, j: (i, 0, 0)),
            pl.BlockSpec((1, 1, H), lambda i, j: (i, 0, 0)),
        ],
        out_specs=[
            pl.BlockSpec((1, BR, 8), lambda i, j: (i, j, 0)),
            pl.BlockSpec((1, BR, 8), lambda i, j: (i, j, 0)),
        ],
        out_shape=[
            jax.ShapeDtypeStruct((NET, NPAD, 8), jnp.float32),
            jax.ShapeDtypeStruct((NET, NPAD, 8), jnp.float32),
        ],
    )(agg, dinv, self1, b1r, gr, ber, w2d, sum1, sum2)


def _tc_d_body(aggd_ref, sd_ref, dinv_ref, b2d_ref, moet_ref, gwt_ref,
               gb_ref, out_ref):
    wl = jnp.dot(gwt_ref[...], moet_ref[...],
                 preferred_element_type=jnp.float32) + gb_ref[...]
    nid = lax.broadcasted_iota(jnp.int32, (8, 1), 0)
    wl = jnp.where(nid < NET, wl, -1e30)
    wm = jnp.max(wl, axis=0, keepdims=True)
    we = jnp.exp(wl - wm)
    w = (we / jnp.sum(we, axis=0, keepdims=True))[0:NET]

    d = dinv_ref[...] * aggd_ref[...] + sd_ref[...] + b2d_ref[...]
    t = jnp.log(1.0 + jnp.exp(-jnp.abs(d)))
    lp1 = jnp.minimum(d, 0.0) - t
    lp0 = -jnp.maximum(d, 0.0) - t
    r0 = jnp.sum(w * lp0, axis=0, keepdims=True)
    r1 = jnp.sum(w * lp1, axis=0, keepdims=True)
    q0 = jnp.sum(w * jnp.exp(lp0), axis=0, keepdims=True)
    q1 = jnp.sum(w * jnp.exp(lp1), axis=0, keepdims=True)
    out_ref[...] = jnp.concatenate(
        [r0, r1, q0, q1, jnp.zeros((4, w.shape[1]), jnp.float32)], axis=0)


def _tc_d(aggd, selfd, dinv2, b2d, moeT, gWT8, gb8):
    return pl.pallas_call(
        _tc_d_body,
        out_shape=jax.ShapeDtypeStruct((8, NPAD), jnp.float32),
    )(aggd, selfd, dinv2, b2d, moeT, gWT8, gb8)


def kernel(features, moe_features, networks, flatten, W1, b1, gamma, beta,
           W2, b2, gW, gb):
    f32 = jnp.float32
    src = networks[:, 0, :].astype(jnp.int32)
    dst = networks[:, 1, :].astype(jnp.int32)
    pad = EPAD - E
    srcp = jnp.pad(src, ((0, 0), (0, pad)), constant_values=DUMP)
    dstp = jnp.pad(dst, ((0, 0), (0, pad)), constant_values=DUMP)
    srcg = (srcp
            + (jnp.arange(NET, dtype=jnp.int32) * NPAD)[:, None]).reshape(-1)
    dst3 = dstp.reshape(NET * NROW, 128)
    flatten_p = jnp.pad(flatten, ((0, NPAD - N), (0, 0)))
    moeT = jnp.pad(moe_features, ((0, NPAD - N), (0, 0))).T
    w2d = (W2[:, :, 1] - W2[:, :, 0]).reshape(NET, H, 1)
    w2d = jnp.pad(w2d, ((0, 0), (0, 0), (0, 7)))
    b2d = (b2[:, 1] - b2[:, 0]).reshape(NET, 1)
    gWT8 = jnp.pad(gW.T, ((0, 4), (0, 0)))
    gb8 = jnp.pad(gb, (0, 4)).reshape(8, 1)
    zeros64 = jnp.zeros((RPT, H), f32)
    zeros8 = jnp.zeros((RPT, 8), f32)

    deg = _sc_deg(dst3).reshape(NET, 1, NPAD)
    z3, dinv, self1 = _tc_b(flatten_p, W1, deg)
    agg1 = _sc_agg64(z3.reshape(NET * NPAD, H), srcg, dst3, zeros64)
    agg1 = agg1.reshape(NET, NPAD, H)
    b1r = b1.reshape(NET, 1, H)
    sum1, sum2 = _tc_c1(agg1, dinv, self1, b1r)
    zd, sd = _tc_c2(agg1, dinv, self1, b1r, gamma.reshape(NET, 1, H),
                    beta.reshape(NET, 1, H), w2d, sum1, sum2)
    aggd = _sc_agg8(zd.reshape(NET * NPAD, 8), srcg, dst3, zeros8)
    aggd = aggd.reshape(NET, NPAD, 8)[:, :, 0]
    outd = _tc_d(aggd, sd[:, :, 0], dinv.reshape(NET, NPAD), b2d, moeT,
                 gWT8, gb8)
    return (jnp.stack([outd[0, :N], outd[1, :N]], axis=1),
            jnp.stack([outd[2, :N], outd[3, :N]], axis=1))


# speedup vs baseline: 1.3404x; 1.2202x over previous
import functools

import jax
import jax.numpy as jnp
from jax import lax
from jax.experimental import pallas as pl
from jax.experimental.pallas import tpu as pltpu
from jax.experimental.pallas import tpu_sc as plsc

N = 10000
E = 320000
NET = 4
D_IN = 128
H = 64

NPAD = 10240
EPAD = 327680
EPT = EPAD // 16
RPT = NPAD // 16
NROW = EPAD // 128
RWT = NROW // 16
DUMP = N
KB1 = 512
KB2 = 2048
BR = 2048
NB = NPAD // BR

_MESH = plsc.VectorSubcoreMesh(core_axis_name="c", subcore_axis_name="s",
                               num_cores=2, num_subcores=16)
_SC_PARAMS = pltpu.CompilerParams(use_tc_tiling_on_sc=False)
_SC_REG_PARAMS = pltpu.CompilerParams(use_tc_tiling_on_sc=False,
                                      needs_layout_passes=False)


def _make_sc_reg(with_gather):
    scratch = [
        pltpu.VMEM((EPT,), jnp.int32),
        pltpu.VMEM((NPAD,), jnp.float32),
    ]
    if with_gather:
        scratch += [
            pltpu.VMEM((EPT,), jnp.int32),
            pltpu.VMEM((NPAD,), jnp.float32),
        ]

    def body(refs):
        if with_gather:
            zd, srcu, dstu, out, didxl, acc, sidxl, zloc = refs
        else:
            dstu, out, didxl, acc = refs
        c = lax.axis_index("c")
        s = lax.axis_index("s")
        ones = jnp.full((16,), 1.0, jnp.float32)
        zeros = jnp.zeros((16,), jnp.float32)
        for nl in range(2):
            net = 2 * c + nl
            ebase = net * EPAD + s * EPT
            pltpu.sync_copy(dstu.at[pl.ds(ebase, EPT)], didxl)
            if with_gather:
                pltpu.sync_copy(srcu.at[pl.ds(ebase, EPT)], sidxl)
                pltpu.sync_copy(zd.at[pl.ds(net * NPAD, NPAD)], zloc)

            def zz(i, _):
                acc[pl.ds(i * 16, 16)] = zeros
                return 0
            lax.fori_loop(0, NPAD // 16, zz, 0)

            def ed(i, _):
                d16 = didxl[pl.ds(i * 16, 16)]
                if with_gather:
                    v16 = plsc.load_gather(zloc, [sidxl[pl.ds(i * 16, 16)]])
                else:
                    v16 = ones
                plsc.addupdate_scatter(acc, [d16], v16)
                return 0
            lax.fori_loop(0, EPT // 16, ed, 0)
            pltpu.sync_copy(
                acc, out.at[pl.ds((net * 16 + s) * NPAD, NPAD)])

    if with_gather:
        def kern(zd, srcu, dstu, out, didxl, acc, sidxl, zloc):
            body((zd, srcu, dstu, out, didxl, acc, sidxl, zloc))
    else:
        def kern(dstu, out, didxl, acc):
            body((dstu, out, didxl, acc))

    return functools.partial(
        pl.kernel,
        out_type=jax.ShapeDtypeStruct((NET * 16 * NPAD,), jnp.float32),
        mesh=_MESH,
        compiler_params=_SC_REG_PARAMS,
        scratch_types=scratch,
    )(kern)


_sc_deg = _make_sc_reg(False)
_sc_agg1 = _make_sc_reg(True)


def _make_sc_agg(width, kb):
    nrow_b = kb // 128
    nbatch = EPT // kb
    npair = nbatch // 2

    @functools.partial(
        pl.kernel,
        out_type=jax.ShapeDtypeStruct((NET * NPAD, width), jnp.float32),
        mesh=_MESH,
        compiler_params=_SC_PARAMS,
        scratch_types=[
            pltpu.VMEM((kb,), jnp.int32),
            pltpu.VMEM((kb,), jnp.int32),
            pltpu.VMEM((nrow_b, 128), jnp.int32),
            pltpu.VMEM((nrow_b, 128), jnp.int32),
            pltpu.VMEM((kb, width), jnp.float32),
            pltpu.VMEM((kb, width), jnp.float32),
            pltpu.VMEM_SHARED((NPAD, width), jnp.float32),
            pltpu.SemaphoreType.DMA,
            pltpu.SemaphoreType.DMA,
            pltpu.SemaphoreType.DMA,
            pltpu.SemaphoreType.DMA,
        ],
    )
    def _sc_agg(z, srcg, dst3, zrows, out, sidx0, sidx1, didx0, didx1,
                rows0, rows1, acc, gs0, gs1, ss0, ss1):
        c = lax.axis_index("c")
        s = lax.axis_index("s")
        slots = ((sidx0, didx0, rows0, gs0, ss0),
                 (sidx1, didx1, rows1, gs1, ss1))

        for nl in range(2):
            net = 2 * c + nl
            pltpu.sync_copy(zrows, acc.at[pl.ds(s * RPT, RPT)])
            plsc.subcore_barrier()
            ebase = net * EPAD + s * EPT
            rbase = net * NROW + s * RWT

            pltpu.sync_copy(srcg.at[pl.ds(ebase, kb)], sidx0)
            pltpu.async_copy(z.at[sidx0], rows0, gs0)

            def pair(p, _):
                for k in range(2):
                    b = 2 * p + k
                    sidx, didx, rows, gsem, _ = slots[k]
                    osidx, _, orows, ogsem, _ = slots[1 - k]
                    @pl.when(b + 1 < nbatch)
                    def _():
                        pltpu.sync_copy(
                            srcg.at[pl.ds(ebase + (b + 1) * kb, kb)], osidx)
                        pltpu.async_copy(z.at[osidx], orows, ogsem)
                    pltpu.make_async_copy(z.at[sidx], rows, gsem).wait()
                    pltpu.sync_copy(
                        dst3.at[pl.ds(rbase + b * nrow_b, nrow_b)], didx)
                    for j in range(nrow_b):
                        pltpu.sync_copy(rows.at[pl.ds(j * 128, 128)],
                                        acc.at[didx.at[j]], add=True)
                return 0
            lax.fori_loop(0, npair, pair, 0)
            plsc.subcore_barrier()
            pltpu.sync_copy(acc.at[pl.ds(s * RPT, RPT)],
                            out.at[pl.ds(net * NPAD + s * RPT, RPT)])
            plsc.subcore_barrier()

    return _sc_agg


_sc_agg64 = _make_sc_agg(H, KB1)


def _tc_b_body(f_ref, w1_ref, deg_ref, z_ref, dinv_ref, s1_ref):
    y = jnp.dot(f_ref[...], w1_ref[0], preferred_element_type=jnp.float32)
    dv = lax.rsqrt(jnp.sum(deg_ref[0], axis=0) + 2.0)
    z_ref[0] = dv[:, None] * y
    dinv_ref[0, 0] = dv
    s1_ref[0] = (2.0 * dv * dv)[:, None] * y


def _tc_b(flatten_p, W1, deg):
    return pl.pallas_call(
        _tc_b_body,
        grid=(NET, NB),
        in_specs=[
            pl.BlockSpec((BR, D_IN), lambda i, j: (j, 0)),
            pl.BlockSpec((1, D_IN, H), lambda i, j: (i, 0, 0)),
            pl.BlockSpec((1, 16, BR), lambda i, j: (i, 0, j)),
        ],
        out_specs=[
            pl.BlockSpec((1, BR, H), lambda i, j: (i, j, 0)),
            pl.BlockSpec((1, 1, BR), lambda i, j: (i, 0, j)),
            pl.BlockSpec((1, BR, H), lambda i, j: (i, j, 0)),
        ],
        out_shape=[
            jax.ShapeDtypeStruct((NET, NPAD, H), jnp.float32),
            jax.ShapeDtypeStruct((NET, 1, NPAD), jnp.float32),
            jax.ShapeDtypeStruct((NET, NPAD, H), jnp.float32),
        ],
    )(flatten_p, W1, deg)


def _x_block(agg_ref, dinv_ref, s1_ref, b1_ref):
    dv = dinv_ref[0, 0]
    x = dv[:, None] * agg_ref[0] + s1_ref[0] + b1_ref[0, 0][None, :]
    return jnp.maximum(x, 0.0), dv


def _tc_c1_body(agg_ref, dinv_ref, s1_ref, b1_ref, sum1_ref, sum2_ref):
    j = pl.program_id(1)
    x, _ = _x_block(agg_ref, dinv_ref, s1_ref, b1_ref)
    rows = lax.broadcasted_iota(jnp.int32, (BR, 1), 0) + j * BR
    mask = (rows < N).astype(jnp.float32)
    xm = x * mask

    @pl.when(j == 0)
    def _():
        sum1_ref[...] = jnp.zeros_like(sum1_ref)
        sum2_ref[...] = jnp.zeros_like(sum2_ref)

    sum1_ref[0, 0] += jnp.sum(xm, axis=0)
    sum2_ref[0, 0] += jnp.sum(xm * x, axis=0)


def _tc_c1(agg, dinv, self1, b1r):
    return pl.pallas_call(
        _tc_c1_body,
        grid=(NET, NB),
        in_specs=[
            pl.BlockSpec((1, BR, H), lambda i, j: (i, j, 0)),
            pl.BlockSpec((1, 1, BR), lambda i, j: (i, 0, j)),
            pl.BlockSpec((1, BR, H), lambda i, j: (i, j, 0)),
            pl.BlockSpec((1, 1, H), lambda i, j: (i, 0, 0)),
        ],
        out_specs=[
            pl.BlockSpec((1, 1, H), lambda i, j: (i, 0, 0)),
            pl.BlockSpec((1, 1, H), lambda i, j: (i, 0, 0)),
        ],
        out_shape=[
            jax.ShapeDtypeStruct((NET, 1, H), jnp.float32),
            jax.ShapeDtypeStruct((NET, 1, H), jnp.float32),
        ],
    )(agg, dinv, self1, b1r)


def _tc_c2_body(agg_ref, dinv_ref, s1_ref, b1_ref, g_ref, be_ref, w2_ref,
                sum1_ref, sum2_ref, zd_ref, sd_ref):
    x, dv = _x_block(agg_ref, dinv_ref, s1_ref, b1_ref)
    mean = sum1_ref[0, 0] / float(N)
    var = sum2_ref[0, 0] / float(N) - mean * mean
    scale = lax.rsqrt(var + 1e-5) * g_ref[0, 0]
    xbn = (x - mean[None, :]) * scale[None, :] + be_ref[0, 0][None, :]
    xd = jnp.dot(xbn, w2_ref[0], preferred_element_type=jnp.float32)
    zd_ref[0] = dv[:, None] * xd
    sd_ref[0] = (2.0 * dv * dv)[:, None] * xd


def _tc_c2(agg, dinv, self1, b1r, gr, ber, w2d, sum1, sum2):
    return pl.pallas_call(
        _tc_c2_body,
        grid=(NET, NB),
        in_specs=[
            pl.BlockSpec((1, BR, H), lambda i, j: (i, j, 0)),
            pl.BlockSpec((1, 1, BR), lambda i, j: (i, 0, j)),
            pl.BlockSpec((1, BR, H), lambda i, j: (i, j, 0)),
            pl.BlockSpec((1, 1, H), lambda i, j: (i, 0, 0)),
            pl.BlockSpec((1, 1, H), lambda i, j: (i, 0, 0)),
            pl.BlockSpec((1, 1, H), lambda i, j: (i, 0, 0)),
            pl.BlockSpec((1, H, 8), lambda i, j: (i, 0, 0)),
            pl.BlockSpec((1, 1, H), lambda i, j: (i, 0, 0)),
            pl.BlockSpec((1, 1, H), lambda i, j: (i, 0, 0)),
        ],
        out_specs=[
            pl.BlockSpec((1, BR, 8), lambda i, j: (i, j, 0)),
            pl.BlockSpec((1, BR, 8), lambda i, j: (i, j, 0)),
        ],
        out_shape=[
            jax.ShapeDtypeStruct((NET, NPAD, 8), jnp.float32),
            jax.ShapeDtypeStruct((NET, NPAD, 8), jnp.float32),
        ],
    )(agg, dinv, self1, b1r, gr, ber, w2d, sum1, sum2)


def _tc_d_body(aggdp_ref, sd_ref, dinv_ref, b2d_ref, moet_ref, gwt_ref,
               gb_ref, out_ref):
    aggd_ref = jnp.sum(aggdp_ref[...], axis=1)
    wl = jnp.dot(gwt_ref[...], moet_ref[...],
                 preferred_element_type=jnp.float32) + gb_ref[...]
    nid = lax.broadcasted_iota(jnp.int32, (8, 1), 0)
    wl = jnp.where(nid < NET, wl, -1e30)
    wm = jnp.max(wl, axis=0, keepdims=True)
    we = jnp.exp(wl - wm)
    w = (we / jnp.sum(we, axis=0, keepdims=True))[0:NET]

    d = dinv_ref[...] * aggd_ref[...] + sd_ref[...] + b2d_ref[...]
    t = jnp.log(1.0 + jnp.exp(-jnp.abs(d)))
    lp1 = jnp.minimum(d, 0.0) - t
    lp0 = -jnp.maximum(d, 0.0) - t
    r0 = jnp.sum(w * lp0, axis=0, keepdims=True)
    r1 = jnp.sum(w * lp1, axis=0, keepdims=True)
    q0 = jnp.sum(w * jnp.exp(lp0), axis=0, keepdims=True)
    q1 = jnp.sum(w * jnp.exp(lp1), axis=0, keepdims=True)
    out_ref[...] = jnp.concatenate(
        [r0, r1, q0, q1, jnp.zeros((4, w.shape[1]), jnp.float32)], axis=0)


def _tc_d(aggd, selfd, dinv2, b2d, moeT, gWT8, gb8):
    return pl.pallas_call(
        _tc_d_body,
        out_shape=jax.ShapeDtypeStruct((8, NPAD), jnp.float32),
    )(aggd, selfd, dinv2, b2d, moeT, gWT8, gb8)


def kernel(features, moe_features, networks, flatten, W1, b1, gamma, beta,
           W2, b2, gW, gb):
    f32 = jnp.float32
    src = networks[:, 0, :].astype(jnp.int32)
    dst = networks[:, 1, :].astype(jnp.int32)
    pad = EPAD - E
    srcp = jnp.pad(src, ((0, 0), (0, pad)), constant_values=DUMP)
    dstp = jnp.pad(dst, ((0, 0), (0, pad)), constant_values=DUMP)
    srcg = (srcp
            + (jnp.arange(NET, dtype=jnp.int32) * NPAD)[:, None]).reshape(-1)
    dst3 = dstp.reshape(NET * NROW, 128)
    flatten_p = jnp.pad(flatten, ((0, NPAD - N), (0, 0)))
    moeT = jnp.pad(moe_features, ((0, NPAD - N), (0, 0))).T
    w2d = (W2[:, :, 1] - W2[:, :, 0]).reshape(NET, H, 1)
    w2d = jnp.pad(w2d, ((0, 0), (0, 0), (0, 7)))
    b2d = (b2[:, 1] - b2[:, 0]).reshape(NET, 1)
    gWT8 = jnp.pad(gW.T, ((0, 4), (0, 0)))
    gb8 = jnp.pad(gb, (0, 4)).reshape(8, 1)
    zeros64 = jnp.zeros((RPT, H), f32)
    srcu = srcp.reshape(-1)
    dstu = dstp.reshape(-1)

    deg = _sc_deg(dstu).reshape(NET, 16, NPAD)
    z3, dinv, self1 = _tc_b(flatten_p, W1, deg)
    agg1 = _sc_agg64(z3.reshape(NET * NPAD, H), srcg, dst3, zeros64)
    agg1 = agg1.reshape(NET, NPAD, H)
    b1r = b1.reshape(NET, 1, H)
    sum1, sum2 = _tc_c1(agg1, dinv, self1, b1r)
    zd, sd = _tc_c2(agg1, dinv, self1, b1r, gamma.reshape(NET, 1, H),
                    beta.reshape(NET, 1, H), w2d, sum1, sum2)
    zdflat = zd[:, :, 0].reshape(NET * NPAD)
    aggdp = _sc_agg1(zdflat, srcu, dstu).reshape(NET, 16, NPAD)
    outd = _tc_d(aggdp, sd[:, :, 0], dinv.reshape(NET, NPAD), b2d, moeT,
                 gWT8, gb8)
    return (jnp.stack([outd[0, :N], outd[1, :N]], axis=1),
            jnp.stack([outd[2, :N], outd[3, :N]], axis=1))


# speedup vs baseline: 1.3888x; 1.0361x over previous
import functools

import jax
import jax.numpy as jnp
from jax import lax
from jax.experimental import pallas as pl
from jax.experimental.pallas import tpu as pltpu
from jax.experimental.pallas import tpu_sc as plsc

N = 10000
E = 320000
NET = 4
D_IN = 128
H = 64

NPAD = 10240
EPAD = 327680
EPT = EPAD // 16
RPT = NPAD // 16
NROW = EPAD // 128
RWT = NROW // 16
DUMP = N
KB1 = 512
KB2 = 2048
BR = 2048
NB = NPAD // BR

_MESH = plsc.VectorSubcoreMesh(core_axis_name="c", subcore_axis_name="s",
                               num_cores=2, num_subcores=16)
_SC_PARAMS = pltpu.CompilerParams(use_tc_tiling_on_sc=False)
_SC_REG_PARAMS = pltpu.CompilerParams(use_tc_tiling_on_sc=False,
                                      needs_layout_passes=False)


def _make_sc_reg(with_gather):
    scratch = [
        pltpu.VMEM((EPT,), jnp.int32),
        pltpu.VMEM((NPAD,), jnp.float32),
    ]
    if with_gather:
        scratch += [
            pltpu.VMEM((EPT,), jnp.int32),
            pltpu.VMEM((NPAD,), jnp.float32),
        ]

    def body(refs):
        if with_gather:
            zd, srcu, dstu, out, didxl, acc, sidxl, zloc = refs
        else:
            dstu, out, didxl, acc = refs
        c = lax.axis_index("c")
        s = lax.axis_index("s")
        ones = jnp.full((16,), 1.0, jnp.float32)
        zeros = jnp.zeros((16,), jnp.float32)
        for nl in range(2):
            net = 2 * c + nl
            ebase = net * EPAD + s * EPT
            pltpu.sync_copy(dstu.at[pl.ds(ebase, EPT)], didxl)
            if with_gather:
                pltpu.sync_copy(srcu.at[pl.ds(ebase, EPT)], sidxl)
                pltpu.sync_copy(zd.at[pl.ds(net * NPAD, NPAD)], zloc)

            def zz(i, _):
                acc[pl.ds(i * 16, 16)] = zeros
                return 0
            lax.fori_loop(0, NPAD // 16, zz, 0)

            def ed(i, _):
                d16 = didxl[pl.ds(i * 16, 16)]
                if with_gather:
                    v16 = plsc.load_gather(zloc, [sidxl[pl.ds(i * 16, 16)]])
                else:
                    v16 = ones
                plsc.addupdate_scatter(acc, [d16], v16)
                return 0
            lax.fori_loop(0, EPT // 16, ed, 0)
            pltpu.sync_copy(
                acc, out.at[pl.ds((net * 16 + s) * NPAD, NPAD)])

    if with_gather:
        def kern(zd, srcu, dstu, out, didxl, acc, sidxl, zloc):
            body((zd, srcu, dstu, out, didxl, acc, sidxl, zloc))
    else:
        def kern(dstu, out, didxl, acc):
            body((dstu, out, didxl, acc))

    return functools.partial(
        pl.kernel,
        out_type=jax.ShapeDtypeStruct((NET * 16 * NPAD,), jnp.float32),
        mesh=_MESH,
        compiler_params=_SC_REG_PARAMS,
        scratch_types=scratch,
    )(kern)


_sc_deg = _make_sc_reg(False)
_sc_agg1 = _make_sc_reg(True)


def _make_sc_agg(width, kb):
    nrow_b = kb // 128
    nbatch = EPT // kb
    npair = nbatch // 2

    @functools.partial(
        pl.kernel,
        out_type=jax.ShapeDtypeStruct((NET * NPAD, width), jnp.float32),
        mesh=_MESH,
        compiler_params=_SC_PARAMS,
        scratch_types=[
            pltpu.VMEM((kb,), jnp.int32),
            pltpu.VMEM((kb,), jnp.int32),
            pltpu.VMEM((nrow_b, 128), jnp.int32),
            pltpu.VMEM((nrow_b, 128), jnp.int32),
            pltpu.VMEM((kb, width), jnp.float32),
            pltpu.VMEM((kb, width), jnp.float32),
            pltpu.VMEM_SHARED((NPAD, width), jnp.float32),
            pltpu.SemaphoreType.DMA,
            pltpu.SemaphoreType.DMA,
            pltpu.SemaphoreType.DMA,
            pltpu.SemaphoreType.DMA,
        ],
    )
    def _sc_agg(z, srcg, dst3, zrows, out, sidx0, sidx1, didx0, didx1,
                rows0, rows1, acc, gs0, gs1, ss0, ss1):
        c = lax.axis_index("c")
        s = lax.axis_index("s")
        slots = ((sidx0, didx0, rows0, gs0, ss0),
                 (sidx1, didx1, rows1, gs1, ss1))

        for nl in range(2):
            net = 2 * c + nl
            pltpu.sync_copy(zrows, acc.at[pl.ds(s * RPT, RPT)])
            plsc.subcore_barrier()
            ebase = net * EPAD + s * EPT
            rbase = net * NROW + s * RWT

            pltpu.sync_copy(srcg.at[pl.ds(ebase, kb)], sidx0)
            pltpu.async_copy(z.at[sidx0], rows0, gs0)

            def pair(p, _):
                for k in range(2):
                    b = 2 * p + k
                    sidx, didx, rows, gsem, _ = slots[k]
                    osidx, _, orows, ogsem, _ = slots[1 - k]
                    @pl.when(b + 1 < nbatch)
                    def _():
                        pltpu.sync_copy(
                            srcg.at[pl.ds(ebase + (b + 1) * kb, kb)], osidx)
                        pltpu.async_copy(z.at[osidx], orows, ogsem)
                    pltpu.make_async_copy(z.at[sidx], rows, gsem).wait()
                    pltpu.sync_copy(
                        dst3.at[pl.ds(rbase + b * nrow_b, nrow_b)], didx)
                    for j in range(nrow_b):
                        pltpu.sync_copy(rows.at[pl.ds(j * 128, 128)],
                                        acc.at[didx.at[j]], add=True)
                return 0
            lax.fori_loop(0, npair, pair, 0)
            plsc.subcore_barrier()
            pltpu.sync_copy(acc.at[pl.ds(s * RPT, RPT)],
                            out.at[pl.ds(net * NPAD + s * RPT, RPT)])
            plsc.subcore_barrier()

    return _sc_agg


_sc_agg64 = _make_sc_agg(H, KB1)


def _tc_b0_body(f_ref, w1_ref, y_ref):
    y_ref[...] = jnp.dot(f_ref[...], w1_ref[...],
                         preferred_element_type=jnp.float32)


def _tc_b0(flatten_p, W1cat):
    return pl.pallas_call(
        _tc_b0_body,
        grid=(NB,),
        in_specs=[
            pl.BlockSpec((BR, D_IN), lambda j: (j, 0)),
            pl.BlockSpec((D_IN, NET * H), lambda j: (0, 0)),
        ],
        out_specs=pl.BlockSpec((BR, NET * H), lambda j: (j, 0)),
        out_shape=jax.ShapeDtypeStruct((NPAD, NET * H), jnp.float32),
    )(flatten_p, W1cat)


def _tc_b1_body(y_ref, deg_ref, z_ref, dinv_ref, s1_ref):
    for n in range(NET):
        dv = lax.rsqrt(jnp.sum(deg_ref[n], axis=0) + 2.0)
        y = y_ref[:, n * H:(n + 1) * H]
        z_ref[n] = dv[:, None] * y
        dinv_ref[n, 0] = dv
        s1_ref[n] = (2.0 * dv * dv)[:, None] * y


def _tc_b1(y1, deg):
    return pl.pallas_call(
        _tc_b1_body,
        grid=(NB,),
        in_specs=[
            pl.BlockSpec((BR, NET * H), lambda j: (j, 0)),
            pl.BlockSpec((NET, 16, BR), lambda j: (0, 0, j)),
        ],
        out_specs=[
            pl.BlockSpec((NET, BR, H), lambda j: (0, j, 0)),
            pl.BlockSpec((NET, 1, BR), lambda j: (0, 0, j)),
            pl.BlockSpec((NET, BR, H), lambda j: (0, j, 0)),
        ],
        out_shape=[
            jax.ShapeDtypeStruct((NET, NPAD, H), jnp.float32),
            jax.ShapeDtypeStruct((NET, 1, NPAD), jnp.float32),
            jax.ShapeDtypeStruct((NET, NPAD, H), jnp.float32),
        ],
    )(y1, deg)


def _x_block(agg_ref, dinv_ref, s1_ref, b1_ref):
    dv = dinv_ref[0, 0]
    x = dv[:, None] * agg_ref[0] + s1_ref[0] + b1_ref[0, 0][None, :]
    return jnp.maximum(x, 0.0), dv


def _tc_c1_body(agg_ref, dinv_ref, s1_ref, b1_ref, sum1_ref, sum2_ref):
    j = pl.program_id(1)
    x, _ = _x_block(agg_ref, dinv_ref, s1_ref, b1_ref)
    rows = lax.broadcasted_iota(jnp.int32, (BR, 1), 0) + j * BR
    mask = (rows < N).astype(jnp.float32)
    xm = x * mask

    @pl.when(j == 0)
    def _():
        sum1_ref[...] = jnp.zeros_like(sum1_ref)
        sum2_ref[...] = jnp.zeros_like(sum2_ref)

    sum1_ref[0, 0] += jnp.sum(xm, axis=0)
    sum2_ref[0, 0] += jnp.sum(xm * x, axis=0)


def _tc_c1(agg, dinv, self1, b1r):
    return pl.pallas_call(
        _tc_c1_body,
        grid=(NET, NB),
        in_specs=[
            pl.BlockSpec((1, BR, H), lambda i, j: (i, j, 0)),
            pl.BlockSpec((1, 1, BR), lambda i, j: (i, 0, j)),
            pl.BlockSpec((1, BR, H), lambda i, j: (i, j, 0)),
            pl.BlockSpec((1, 1, H), lambda i, j: (i, 0, 0)),
        ],
        out_specs=[
            pl.BlockSpec((1, 1, H), lambda i, j: (i, 0, 0)),
            pl.BlockSpec((1, 1, H), lambda i, j: (i, 0, 0)),
        ],
        out_shape=[
            jax.ShapeDtypeStruct((NET, 1, H), jnp.float32),
            jax.ShapeDtypeStruct((NET, 1, H), jnp.float32),
        ],
    )(agg, dinv, self1, b1r)


def _tc_c2_body(agg_ref, dinv_ref, s1_ref, b1_ref, g_ref, be_ref, w2_ref,
                sum1_ref, sum2_ref, zd_ref, sd_ref):
    x, dv = _x_block(agg_ref, dinv_ref, s1_ref, b1_ref)
    mean = sum1_ref[0, 0] / float(N)
    var = sum2_ref[0, 0] / float(N) - mean * mean
    scale = lax.rsqrt(var + 1e-5) * g_ref[0, 0]
    xbn = (x - mean[None, :]) * scale[None, :] + be_ref[0, 0][None, :]
    xd = jnp.dot(xbn, w2_ref[0], preferred_element_type=jnp.float32)
    zd_ref[0] = dv[:, None] * xd
    sd_ref[0] = (2.0 * dv * dv)[:, None] * xd


def _tc_c2(agg, dinv, self1, b1r, gr, ber, w2d, sum1, sum2):
    return pl.pallas_call(
        _tc_c2_body,
        grid=(NET, NB),
        in_specs=[
            pl.BlockSpec((1, BR, H), lambda i, j: (i, j, 0)),
            pl.BlockSpec((1, 1, BR), lambda i, j: (i, 0, j)),
            pl.BlockSpec((1, BR, H), lambda i, j: (i, j, 0)),
            pl.BlockSpec((1, 1, H), lambda i, j: (i, 0, 0)),
            pl.BlockSpec((1, 1, H), lambda i, j: (i, 0, 0)),
            pl.BlockSpec((1, 1, H), lambda i, j: (i, 0, 0)),
            pl.BlockSpec((1, H, 8), lambda i, j: (i, 0, 0)),
            pl.BlockSpec((1, 1, H), lambda i, j: (i, 0, 0)),
            pl.BlockSpec((1, 1, H), lambda i, j: (i, 0, 0)),
        ],
        out_specs=[
            pl.BlockSpec((1, BR, 8), lambda i, j: (i, j, 0)),
            pl.BlockSpec((1, BR, 8), lambda i, j: (i, j, 0)),
        ],
        out_shape=[
            jax.ShapeDtypeStruct((NET, NPAD, 8), jnp.float32),
            jax.ShapeDtypeStruct((NET, NPAD, 8), jnp.float32),
        ],
    )(agg, dinv, self1, b1r, gr, ber, w2d, sum1, sum2)


def _tc_d_body(aggdp_ref, sd_ref, dinv_ref, b2d_ref, moet_ref, gwt_ref,
               gb_ref, out_ref):
    aggd_ref = jnp.sum(aggdp_ref[...], axis=1)
    wl = jnp.dot(gwt_ref[...], moet_ref[...],
                 preferred_element_type=jnp.float32) + gb_ref[...]
    nid = lax.broadcasted_iota(jnp.int32, (8, 1), 0)
    wl = jnp.where(nid < NET, wl, -1e30)
    wm = jnp.max(wl, axis=0, keepdims=True)
    we = jnp.exp(wl - wm)
    w = (we / jnp.sum(we, axis=0, keepdims=True))[0:NET]

    d = dinv_ref[...] * aggd_ref[...] + sd_ref[...] + b2d_ref[...]
    t = jnp.log(1.0 + jnp.exp(-jnp.abs(d)))
    lp1 = jnp.minimum(d, 0.0) - t
    lp0 = -jnp.maximum(d, 0.0) - t
    r0 = jnp.sum(w * lp0, axis=0, keepdims=True)
    r1 = jnp.sum(w * lp1, axis=0, keepdims=True)
    q0 = jnp.sum(w * jnp.exp(lp0), axis=0, keepdims=True)
    q1 = jnp.sum(w * jnp.exp(lp1), axis=0, keepdims=True)
    out_ref[...] = jnp.concatenate(
        [r0, r1, q0, q1, jnp.zeros((4, w.shape[1]), jnp.float32)], axis=0)


def _tc_d(aggd, selfd, dinv2, b2d, moeT, gWT8, gb8):
    return pl.pallas_call(
        _tc_d_body,
        out_shape=jax.ShapeDtypeStruct((8, NPAD), jnp.float32),
    )(aggd, selfd, dinv2, b2d, moeT, gWT8, gb8)


def kernel(features, moe_features, networks, flatten, W1, b1, gamma, beta,
           W2, b2, gW, gb):
    f32 = jnp.float32
    src = networks[:, 0, :].astype(jnp.int32)
    dst = networks[:, 1, :].astype(jnp.int32)
    pad = EPAD - E
    srcp = jnp.pad(src, ((0, 0), (0, pad)), constant_values=DUMP)
    dstp = jnp.pad(dst, ((0, 0), (0, pad)), constant_values=DUMP)
    srcg = (srcp
            + (jnp.arange(NET, dtype=jnp.int32) * NPAD)[:, None]).reshape(-1)
    dst3 = dstp.reshape(NET * NROW, 128)
    flatten_p = jnp.pad(flatten, ((0, NPAD - N), (0, 0)))
    moeT = jnp.pad(moe_features, ((0, NPAD - N), (0, 0))).T
    w2d = (W2[:, :, 1] - W2[:, :, 0]).reshape(NET, H, 1)
    w2d = jnp.pad(w2d, ((0, 0), (0, 0), (0, 7)))
    b2d = (b2[:, 1] - b2[:, 0]).reshape(NET, 1)
    gWT8 = jnp.pad(gW.T, ((0, 4), (0, 0)))
    gb8 = jnp.pad(gb, (0, 4)).reshape(8, 1)
    zeros64 = jnp.zeros((RPT, H), f32)
    srcu = srcp.reshape(-1)
    dstu = dstp.reshape(-1)

    W1cat = W1.transpose(1, 0, 2).reshape(D_IN, NET * H)
    y1 = _tc_b0(flatten_p, W1cat)
    deg = _sc_deg(dstu).reshape(NET, 16, NPAD)
    z3, dinv, self1 = _tc_b1(y1, deg)
    agg1 = _sc_agg64(z3.reshape(NET * NPAD, H), srcg, dst3, zeros64)
    agg1 = agg1.reshape(NET, NPAD, H)
    b1r = b1.reshape(NET, 1, H)
    sum1, sum2 = _tc_c1(agg1, dinv, self1, b1r)
    zd, sd = _tc_c2(agg1, dinv, self1, b1r, gamma.reshape(NET, 1, H),
                    beta.reshape(NET, 1, H), w2d, sum1, sum2)
    zdflat = zd[:, :, 0].reshape(NET * NPAD)
    aggdp = _sc_agg1(zdflat, srcu, dstu).reshape(NET, 16, NPAD)
    outd = _tc_d(aggdp, sd[:, :, 0], dinv.reshape(NET, NPAD), b2d, moeT,
                 gWT8, gb8)
    return (jnp.stack([outd[0, :N], outd[1, :N]], axis=1),
            jnp.stack([outd[2, :N], outd[3, :N]], axis=1))
